# Initial kernel scaffold; baseline (speedup 1.0000x reference)
import jax
import jax.numpy as jnp
from jax.experimental import pallas as pl


def kernel(x, Wr, W1, W2):
    raise NotImplementedError("write your pallas kernel here")


# speedup vs baseline: 1.9211x; 1.9211x over previous
import functools
import math

import jax
import jax.numpy as jnp
from jax import lax
from jax.experimental import pallas as pl
from jax.experimental.pallas import tpu as pltpu
from jax.experimental.pallas import tpu_sc as plsc

T = 2048
D = 768
E = 8
K = 2
F = 2048

TILE = 128
NP = T * K + E * TILE
NT = NP // TILE

NWK = 32
GROWS = NP // NWK
GCH = 80
CROWS = T // NWK


def _router_body(x_ref, wr_ref, idx_ref, prob_ref):
    x = x_ref[...]
    wr = wr_ref[...]
    logits = lax.dot_general(x, wr, (((1,), (1,)), ((), ())),
                             preferred_element_type=jnp.float32)
    iota = lax.broadcasted_iota(jnp.int32, (T, E), 1)
    m1 = jnp.max(logits, axis=1, keepdims=True)
    ex = jnp.exp(logits - m1)
    p = ex / jnp.sum(ex, axis=1, keepdims=True)
    a1 = jnp.min(jnp.where(logits == m1, iota, E), axis=1, keepdims=True)
    p1 = jnp.sum(jnp.where(iota == a1, p, 0.0), axis=1, keepdims=True)
    l2 = jnp.where(iota == a1, -jnp.inf, logits)
    m2 = jnp.max(l2, axis=1, keepdims=True)
    a2 = jnp.min(jnp.where(l2 == m2, iota, E), axis=1, keepdims=True)
    p2 = jnp.sum(jnp.where(iota == a2, p, 0.0), axis=1, keepdims=True)
    s = p1 + p2
    idx_ref[...] = jnp.concatenate([a1, a2], axis=1).astype(jnp.int32)
    prob_ref[...] = jnp.concatenate([p1 / s, p2 / s], axis=1)


def _router(x2d, Wr):
    return pl.pallas_call(
        _router_body,
        out_shape=(
            jax.ShapeDtypeStruct((T, K), jnp.int32),
            jax.ShapeDtypeStruct((T, K), jnp.float32),
        ),
    )(x2d, Wr)


@functools.cache
def _sc_mesh():
    return plsc.VectorSubcoreMesh(core_axis_name="c", subcore_axis_name="s")


@functools.cache
def _sc_gather():
    @functools.partial(
        pl.kernel,
        mesh=_sc_mesh(),
        out_type=jax.ShapeDtypeStruct((NP, D), jnp.float32),
        scratch_types=[
            pltpu.VMEM((GROWS // GCH, GCH), jnp.int32),
            pltpu.VMEM((GROWS, D), jnp.float32),
            pltpu.SemaphoreType.DMA,
        ],
    )
    def gather(x_hbm, idx_hbm, out_hbm, idx_v, rows_v, sem):
        wid = lax.axis_index("s") * 2 + lax.axis_index("c")
        base = wid * (GROWS // GCH)
        pltpu.sync_copy(idx_hbm.at[pl.ds(base, GROWS // GCH)], idx_v)
        for k in range(GROWS // GCH):
            pltpu.async_copy(
                x_hbm.at[idx_v.at[k]], rows_v.at[pl.ds(k * GCH, GCH)], sem
            ).wait()
        pltpu.sync_copy(rows_v, out_hbm.at[pl.ds(wid * GROWS, GROWS)])

    return gather


def _ffn_body(te_ref, xg_ref, w1_ref, w2_ref, p_ref, og_ref):
    xg = xg_ref[...]
    w1 = w1_ref[0]
    w2 = w2_ref[0]
    h = lax.dot_general(xg, w1, (((1,), (1,)), ((), ())),
                        preferred_element_type=jnp.float32)
    h = 0.5 * h * (1.0 + lax.erf(h * (1.0 / math.sqrt(2.0))))
    o = lax.dot_general(h, w2, (((1,), (1,)), ((), ())),
                        preferred_element_type=jnp.float32)
    og_ref[...] = o * p_ref[...]


def _ffn(te, xg, W1, W2, p_pad):
    grid_spec = pltpu.PrefetchScalarGridSpec(
        num_scalar_prefetch=1,
        grid=(NT,),
        in_specs=[
            pl.BlockSpec((TILE, D), lambda i, te: (i, 0)),
            pl.BlockSpec((1, F, D), lambda i, te: (te[i], 0, 0)),
            pl.BlockSpec((1, D, F), lambda i, te: (te[i], 0, 0)),
            pl.BlockSpec((TILE, 1), lambda i, te: (i, 0)),
        ],
        out_specs=pl.BlockSpec((TILE, D), lambda i, te: (i, 0)),
    )
    return pl.pallas_call(
        _ffn_body,
        grid_spec=grid_spec,
        out_shape=jax.ShapeDtypeStruct((NP, D), jnp.float32),
    )(te, xg, W1, W2, p_pad)


@functools.cache
def _sc_combine():
    @functools.partial(
        pl.kernel,
        mesh=_sc_mesh(),
        out_type=jax.ShapeDtypeStruct((T, D), jnp.float32),
        scratch_types=[
            pltpu.VMEM((CROWS,), jnp.int32),
            pltpu.VMEM((CROWS,), jnp.int32),
            pltpu.VMEM((CROWS, D), jnp.float32),
            pltpu.VMEM((CROWS, D), jnp.float32),
            pltpu.SemaphoreType.DMA,
            pltpu.SemaphoreType.DMA,
        ],
    )
    def combine(og_hbm, pos1_hbm, pos2_hbm, out_hbm,
                p1v, p2v, av, bv, sema, semb):
        wid = lax.axis_index("s") * 2 + lax.axis_index("c")
        base = wid * CROWS
        pltpu.sync_copy(pos1_hbm.at[pl.ds(base, CROWS)], p1v)
        pltpu.sync_copy(pos2_hbm.at[pl.ds(base, CROWS)], p2v)
        ca = pltpu.async_copy(og_hbm.at[p1v], av, sema)
        cb = pltpu.async_copy(og_hbm.at[p2v], bv, semb)
        ca.wait()
        cb.wait()

        def row(r, _):
            def col(j, _):
                av[r, pl.ds(j * 16, 16)] = (
                    av[r, pl.ds(j * 16, 16)] + bv[r, pl.ds(j * 16, 16)]
                )
                return 0
            return lax.fori_loop(0, D // 16, col, 0)

        lax.fori_loop(0, CROWS, row, 0)
        pltpu.sync_copy(av, out_hbm.at[pl.ds(base, CROWS)])

    return combine


def _dispatch_plan(idx, prob):
    e_all = jnp.concatenate([idx[:, 0], idx[:, 1]])
    t_all = jnp.concatenate([jnp.arange(T, dtype=jnp.int32)] * 2)
    p_all = jnp.concatenate([prob[:, 0], prob[:, 1]])
    order = jnp.argsort(e_all)
    se = e_all[order]
    counts = jnp.bincount(e_all, length=E)
    start = jnp.concatenate(
        [jnp.zeros(1, jnp.int32), jnp.cumsum(counts)[:-1].astype(jnp.int32)])
    cap = ((counts + TILE - 1) // TILE) * TILE
    end_pad = jnp.cumsum(cap).astype(jnp.int32)
    start_pad = jnp.concatenate([jnp.zeros(1, jnp.int32), end_pad[:-1]])
    r = jnp.arange(2 * T, dtype=jnp.int32) - start[se]
    pos = start_pad[se] + r
    tok_pad = jnp.zeros(NP, jnp.int32).at[pos].set(t_all[order])
    p_pad = jnp.zeros(NP, jnp.float32).at[pos].set(p_all[order])
    posa = jnp.zeros(2 * T, jnp.int32).at[order].set(pos)
    te = jnp.searchsorted(
        end_pad, jnp.arange(NT, dtype=jnp.int32) * TILE, side="right")
    te = jnp.minimum(te, E - 1).astype(jnp.int32)
    return tok_pad, p_pad, te, posa[:T], posa[T:]


def kernel(x, Wr, W1, W2):
    b, s, d = x.shape
    x2d = x.reshape(T, D)
    idx, prob = _router(x2d, Wr)
    tok_pad, p_pad, te, pos1, pos2 = _dispatch_plan(idx, prob)
    xg = _sc_gather()(x2d, tok_pad.reshape(NP // GCH, GCH))
    og = _ffn(te, xg, W1, W2, p_pad.reshape(NP, 1))
    out = _sc_combine()(og, pos1, pos2)
    return out.reshape(b, s, d)


# speedup vs baseline: 3.2228x; 1.6775x over previous
import functools
import math

import jax
import jax.numpy as jnp
from jax import lax
from jax.experimental import pallas as pl
from jax.experimental.pallas import tpu as pltpu
from jax.experimental.pallas import tpu_sc as plsc

T = 2048
D = 768
E = 8
K = 2
F = 2048

TILE = 128
NP = T * K + E * TILE
NT = NP // TILE

NWK = 32
CROWS = T // NWK


def _router_body(x_ref, wr_ref, pos1_ref, pos2_ref, p1_ref, p2_ref, te_ref):
    x = x_ref[...]
    wr = wr_ref[...]
    logits = lax.dot_general(x, wr, (((1,), (1,)), ((), ())),
                             preferred_element_type=jnp.float32)
    iota = lax.broadcasted_iota(jnp.int32, (T, E), 1)
    m1 = jnp.max(logits, axis=1, keepdims=True)
    ex = jnp.exp(logits - m1)
    p = ex / jnp.sum(ex, axis=1, keepdims=True)
    a1 = jnp.min(jnp.where(logits == m1, iota, E), axis=1, keepdims=True)
    p1 = jnp.sum(jnp.where(iota == a1, p, 0.0), axis=1, keepdims=True)
    l2 = jnp.where(iota == a1, -jnp.inf, logits)
    m2 = jnp.max(l2, axis=1, keepdims=True)
    a2 = jnp.min(jnp.where(l2 == m2, iota, E), axis=1, keepdims=True)
    p2 = jnp.sum(jnp.where(iota == a2, p, 0.0), axis=1, keepdims=True)
    s = p1 + p2

    oh1 = (iota == a1).astype(jnp.float32)
    oh2 = (iota == a2).astype(jnp.float32)
    cnt1 = jnp.sum(oh1, axis=0, keepdims=True)
    cnt2 = jnp.sum(oh2, axis=0, keepdims=True)
    counts = cnt1 + cnt2
    def excl_prefix(oh):
        pre = oh
        k = 1
        while k < T:
            pre = pre + jnp.concatenate(
                [jnp.zeros((k, E), oh.dtype), pre[:T - k]], axis=0)
            k *= 2
        return pre - oh

    pre1 = excl_prefix(oh1)
    pre2 = excl_prefix(oh2)
    cap = jnp.floor((counts + (TILE - 1)) * (1.0 / TILE)) * TILE
    tri = (lax.broadcasted_iota(jnp.int32, (E, E), 0)
           <= lax.broadcasted_iota(jnp.int32, (E, E), 1)).astype(jnp.float32)
    end_pad = lax.dot_general(cap, tri, (((1,), (0,)), ((), ())),
                              preferred_element_type=jnp.float32)
    start_pad = end_pad - cap
    b1 = jnp.sum(jnp.where(iota == a1, start_pad + pre1, 0.0),
                 axis=1, keepdims=True)
    b2 = jnp.sum(jnp.where(iota == a2, start_pad + cnt1 + pre2, 0.0),
                 axis=1, keepdims=True)
    pos1_ref[...] = b1.astype(jnp.int32).reshape(NWK, CROWS)
    pos2_ref[...] = b2.astype(jnp.int32).reshape(NWK, CROWS)
    p1_ref[...] = jnp.broadcast_to(p1 / s, (T, 16))
    p2_ref[...] = jnp.broadcast_to(p2 / s, (T, 16))
    ts = lax.broadcasted_iota(jnp.int32, (NT, E), 0) * TILE
    te = jnp.sum((ts >= end_pad.astype(jnp.int32)).astype(jnp.int32),
                 axis=1, keepdims=True)
    te_ref[...] = jnp.minimum(te, E - 1)


def _router(x2d, Wr):
    return pl.pallas_call(
        _router_body,
        out_shape=(
            jax.ShapeDtypeStruct((NWK, CROWS), jnp.int32),
            jax.ShapeDtypeStruct((NWK, CROWS), jnp.int32),
            jax.ShapeDtypeStruct((T, 16), jnp.float32),
            jax.ShapeDtypeStruct((T, 16), jnp.float32),
            jax.ShapeDtypeStruct((NT, 1), jnp.int32),
        ),
    )(x2d, Wr)


@functools.cache
def _sc_mesh():
    return plsc.VectorSubcoreMesh(core_axis_name="c", subcore_axis_name="s")


@functools.cache
def _sc_dispatch():
    @functools.partial(
        pl.kernel,
        mesh=_sc_mesh(),
        out_type=jax.ShapeDtypeStruct((NP, D), jnp.float32),
        scratch_types=[
            pltpu.VMEM((CROWS,), jnp.int32),
            pltpu.VMEM((CROWS,), jnp.int32),
            pltpu.VMEM((CROWS, D), jnp.float32),
            pltpu.SemaphoreType.DMA,
            pltpu.SemaphoreType.DMA,
            pltpu.SemaphoreType.DMA,
        ],
    )
    def dispatch(x_hbm, pos1_hbm, pos2_hbm, xg_hbm, i1v, i2v, rows_v,
                 s0, s1, s2):
        wid = lax.axis_index("s") * 2 + lax.axis_index("c")
        base = wid * CROWS
        pltpu.sync_copy(pos1_hbm.at[wid], i1v)
        pltpu.sync_copy(pos2_hbm.at[wid], i2v)
        pltpu.async_copy(x_hbm.at[pl.ds(base, CROWS)], rows_v, s0).wait()
        c1 = pltpu.async_copy(rows_v, xg_hbm.at[i1v], s1)
        c2 = pltpu.async_copy(rows_v, xg_hbm.at[i2v], s2)
        c1.wait()
        c2.wait()

    return dispatch


def _ffn_body(te_ref, xg_ref, w1_ref, w2_ref, og_ref, w1b, w2b):
    i = pl.program_id(0)
    changed = jnp.logical_or(
        i == 0, te_ref[i, 0] != te_ref[jnp.maximum(i - 1, 0), 0])

    @pl.when(changed)
    def _():
        w1b[...] = w1_ref[0].astype(jnp.bfloat16)
        w2b[...] = w2_ref[0].astype(jnp.bfloat16)

    xg = xg_ref[...].astype(jnp.bfloat16)
    h = lax.dot_general(xg, w1b[...], (((1,), (1,)), ((), ())),
                        preferred_element_type=jnp.float32)
    h = 0.5 * h * (1.0 + lax.erf(h * (1.0 / math.sqrt(2.0))))
    og_ref[...] = lax.dot_general(h.astype(jnp.bfloat16), w2b[...],
                                  (((1,), (1,)), ((), ())),
                                  preferred_element_type=jnp.float32)


def _ffn(te, xg, W1, W2):
    grid_spec = pltpu.PrefetchScalarGridSpec(
        num_scalar_prefetch=1,
        grid=(NT,),
        in_specs=[
            pl.BlockSpec((TILE, D), lambda i, te: (i, 0)),
            pl.BlockSpec((1, F, D), lambda i, te: (te[i, 0], 0, 0)),
            pl.BlockSpec((1, D, F), lambda i, te: (te[i, 0], 0, 0)),
        ],
        out_specs=pl.BlockSpec((TILE, D), lambda i, te: (i, 0)),
        scratch_shapes=[
            pltpu.VMEM((F, D), jnp.bfloat16),
            pltpu.VMEM((D, F), jnp.bfloat16),
        ],
    )
    return pl.pallas_call(
        _ffn_body,
        grid_spec=grid_spec,
        out_shape=jax.ShapeDtypeStruct((NP, D), jnp.float32),
    )(te, xg, W1, W2)


@functools.cache
def _sc_combine():
    @functools.partial(
        pl.kernel,
        mesh=_sc_mesh(),
        out_type=jax.ShapeDtypeStruct((T, D), jnp.float32),
        scratch_types=[
            pltpu.VMEM((CROWS,), jnp.int32),
            pltpu.VMEM((CROWS,), jnp.int32),
            pltpu.VMEM((CROWS, 16), jnp.float32),
            pltpu.VMEM((CROWS, 16), jnp.float32),
            pltpu.VMEM((CROWS, D), jnp.float32),
            pltpu.VMEM((CROWS, D), jnp.float32),
            pltpu.SemaphoreType.DMA,
            pltpu.SemaphoreType.DMA,
        ],
    )
    def combine(og_hbm, i1_hbm, i2_hbm, p1_hbm, p2_hbm, out_hbm,
                i1v, i2v, pv1, pv2, av, bv, sa, sb):
        wid = lax.axis_index("s") * 2 + lax.axis_index("c")
        base = wid * CROWS
        pltpu.sync_copy(i1_hbm.at[wid], i1v)
        pltpu.sync_copy(i2_hbm.at[wid], i2v)
        pltpu.sync_copy(p1_hbm.at[pl.ds(base, CROWS)], pv1)
        pltpu.sync_copy(p2_hbm.at[pl.ds(base, CROWS)], pv2)
        ca = pltpu.async_copy(og_hbm.at[i1v], av, sa)
        cb = pltpu.async_copy(og_hbm.at[i2v], bv, sb)
        ca.wait()
        cb.wait()

        @plsc.parallel_loop(0, CROWS, 1, unroll=4)
        def body(r):
            pa = pv1[r]
            pb = pv2[r]
            for j in range(D // 16):
                av[r, pl.ds(j * 16, 16)] = (
                    av[r, pl.ds(j * 16, 16)] * pa
                    + bv[r, pl.ds(j * 16, 16)] * pb)

        pltpu.sync_copy(av, out_hbm.at[pl.ds(base, CROWS)])

    return combine


def kernel(x, Wr, W1, W2):
    b, s, d = x.shape
    x2d = x.reshape(T, D)
    pos1, pos2, p1b, p2b, te = _router(x2d, Wr)
    xg = _sc_dispatch()(x2d, pos1, pos2)
    og = _ffn(te, xg, W1, W2)
    out = _sc_combine()(og, pos1, pos2, p1b, p2b)
    return out.reshape(b, s, d)


# speedup vs baseline: 3.3753x; 1.0473x over previous
import functools
import math

import jax
import jax.numpy as jnp
from jax import lax
from jax.experimental import pallas as pl
from jax.experimental.pallas import tpu as pltpu
from jax.experimental.pallas import tpu_sc as plsc

T = 2048
D = 768
E = 8
K = 2
F = 2048

TILE = 128
NP = T * K + E * TILE
NT = NP // TILE

NWK = 32
CROWS = T // NWK


def _router_body(x_ref, wr_ref, pos1_ref, pos2_ref, p1_ref, p2_ref, te_ref,
                 nu_ref):
    x = x_ref[...]
    wr = wr_ref[...]
    logits = lax.dot_general(x, wr, (((1,), (1,)), ((), ())),
                             preferred_element_type=jnp.float32)
    iota = lax.broadcasted_iota(jnp.int32, (T, E), 1)
    m1 = jnp.max(logits, axis=1, keepdims=True)
    ex = jnp.exp(logits - m1)
    p = ex / jnp.sum(ex, axis=1, keepdims=True)
    a1 = jnp.min(jnp.where(logits == m1, iota, E), axis=1, keepdims=True)
    p1 = jnp.sum(jnp.where(iota == a1, p, 0.0), axis=1, keepdims=True)
    l2 = jnp.where(iota == a1, -jnp.inf, logits)
    m2 = jnp.max(l2, axis=1, keepdims=True)
    a2 = jnp.min(jnp.where(l2 == m2, iota, E), axis=1, keepdims=True)
    p2 = jnp.sum(jnp.where(iota == a2, p, 0.0), axis=1, keepdims=True)
    s = p1 + p2

    oh1 = (iota == a1).astype(jnp.float32)
    oh2 = (iota == a2).astype(jnp.float32)
    cnt1 = jnp.sum(oh1, axis=0, keepdims=True)
    cnt2 = jnp.sum(oh2, axis=0, keepdims=True)
    counts = cnt1 + cnt2
    def excl_prefix(oh):
        pre = oh
        k = 1
        while k < T:
            pre = pre + jnp.concatenate(
                [jnp.zeros((k, E), oh.dtype), pre[:T - k]], axis=0)
            k *= 2
        return pre - oh

    pre1 = excl_prefix(oh1)
    pre2 = excl_prefix(oh2)
    cap = jnp.floor((counts + (TILE - 1)) * (1.0 / TILE)) * TILE
    tri = (lax.broadcasted_iota(jnp.int32, (E, E), 0)
           <= lax.broadcasted_iota(jnp.int32, (E, E), 1)).astype(jnp.float32)
    end_pad = lax.dot_general(cap, tri, (((1,), (0,)), ((), ())),
                              preferred_element_type=jnp.float32)
    start_pad = end_pad - cap
    b1 = jnp.sum(jnp.where(iota == a1, start_pad + pre1, 0.0),
                 axis=1, keepdims=True)
    b2 = jnp.sum(jnp.where(iota == a2, start_pad + cnt1 + pre2, 0.0),
                 axis=1, keepdims=True)
    pos1_ref[...] = b1.astype(jnp.int32).reshape(NWK, CROWS)
    pos2_ref[...] = b2.astype(jnp.int32).reshape(NWK, CROWS)
    p1_ref[...] = jnp.broadcast_to(p1 / s, (T, 16))
    p2_ref[...] = jnp.broadcast_to(p2 / s, (T, 16))
    ts = lax.broadcasted_iota(jnp.int32, (NT, E), 0) * TILE
    te = jnp.sum((ts >= end_pad.astype(jnp.int32)).astype(jnp.int32),
                 axis=1, keepdims=True)
    te_ref[...] = jnp.minimum(te, E - 1)
    lane = lax.broadcasted_iota(jnp.int32, (1, E), 1)
    total = jnp.sum(jnp.where(lane == E - 1, end_pad, 0.0),
                    axis=1, keepdims=True)
    nu_ref[...] = (total * (1.0 / TILE)).astype(jnp.int32)


def _router(x2d, Wr):
    return pl.pallas_call(
        _router_body,
        out_shape=(
            jax.ShapeDtypeStruct((NWK, CROWS), jnp.int32),
            jax.ShapeDtypeStruct((NWK, CROWS), jnp.int32),
            jax.ShapeDtypeStruct((T, 16), jnp.float32),
            jax.ShapeDtypeStruct((T, 16), jnp.float32),
            jax.ShapeDtypeStruct((NT, 1), jnp.int32),
            jax.ShapeDtypeStruct((1, 1), jnp.int32),
        ),
    )(x2d, Wr)


@functools.cache
def _sc_mesh():
    return plsc.VectorSubcoreMesh(core_axis_name="c", subcore_axis_name="s")


@functools.cache
def _sc_dispatch():
    @functools.partial(
        pl.kernel,
        mesh=_sc_mesh(),
        out_type=jax.ShapeDtypeStruct((NP, D), jnp.float32),
        scratch_types=[
            pltpu.VMEM((CROWS,), jnp.int32),
            pltpu.VMEM((CROWS,), jnp.int32),
            pltpu.VMEM((CROWS, D), jnp.float32),
            pltpu.SemaphoreType.DMA,
            pltpu.SemaphoreType.DMA,
            pltpu.SemaphoreType.DMA,
        ],
    )
    def dispatch(x_hbm, pos1_hbm, pos2_hbm, xg_hbm, i1v, i2v, rows_v,
                 s0, s1, s2):
        wid = lax.axis_index("s") * 2 + lax.axis_index("c")
        base = wid * CROWS
        pltpu.sync_copy(pos1_hbm.at[wid], i1v)
        pltpu.sync_copy(pos2_hbm.at[wid], i2v)
        pltpu.async_copy(x_hbm.at[pl.ds(base, CROWS)], rows_v, s0).wait()
        c1 = pltpu.async_copy(rows_v, xg_hbm.at[i1v], s1)
        c2 = pltpu.async_copy(rows_v, xg_hbm.at[i2v], s2)
        c1.wait()
        c2.wait()

    return dispatch


def _ffn_body(te_ref, nu_ref, xg_ref, w1_ref, w2_ref, og_ref, w1b, w2b):
    i = pl.program_id(0)
    active = i < nu_ref[0, 0]
    changed = jnp.logical_and(
        active,
        jnp.logical_or(i == 0,
                       te_ref[i, 0] != te_ref[jnp.maximum(i - 1, 0), 0]))

    @pl.when(changed)
    def _():
        w1b[...] = w1_ref[0].astype(jnp.bfloat16)
        w2b[...] = w2_ref[0].astype(jnp.bfloat16)

    @pl.when(active)
    def _():
        xg = xg_ref[...].astype(jnp.bfloat16)
        h = lax.dot_general(xg, w1b[...], (((1,), (1,)), ((), ())),
                            preferred_element_type=jnp.float32)
        h = 0.5 * h * (1.0 + lax.erf(h * (1.0 / math.sqrt(2.0))))
        og_ref[...] = lax.dot_general(h.astype(jnp.bfloat16), w2b[...],
                                      (((1,), (1,)), ((), ())),
                                      preferred_element_type=jnp.float32)


def _ffn(te, nu, xg, W1, W2):
    grid_spec = pltpu.PrefetchScalarGridSpec(
        num_scalar_prefetch=2,
        grid=(NT,),
        in_specs=[
            pl.BlockSpec((TILE, D), lambda i, te, nu: (i, 0)),
            pl.BlockSpec((1, F, D), lambda i, te, nu: (te[i, 0], 0, 0)),
            pl.BlockSpec((1, D, F), lambda i, te, nu: (te[i, 0], 0, 0)),
        ],
        out_specs=pl.BlockSpec((TILE, D), lambda i, te, nu: (i, 0)),
        scratch_shapes=[
            pltpu.VMEM((F, D), jnp.bfloat16),
            pltpu.VMEM((D, F), jnp.bfloat16),
        ],
    )
    return pl.pallas_call(
        _ffn_body,
        grid_spec=grid_spec,
        out_shape=jax.ShapeDtypeStruct((NP, D), jnp.float32),
    )(te, nu, xg, W1, W2)


@functools.cache
def _sc_combine():
    @functools.partial(
        pl.kernel,
        mesh=_sc_mesh(),
        out_type=jax.ShapeDtypeStruct((T, D), jnp.float32),
        scratch_types=[
            pltpu.VMEM((CROWS,), jnp.int32),
            pltpu.VMEM((CROWS,), jnp.int32),
            pltpu.VMEM((CROWS, 16), jnp.float32),
            pltpu.VMEM((CROWS, 16), jnp.float32),
            pltpu.VMEM((CROWS, D), jnp.float32),
            pltpu.VMEM((CROWS, D), jnp.float32),
            pltpu.SemaphoreType.DMA,
            pltpu.SemaphoreType.DMA,
        ],
    )
    def combine(og_hbm, i1_hbm, i2_hbm, p1_hbm, p2_hbm, out_hbm,
                i1v, i2v, pv1, pv2, av, bv, sa, sb):
        wid = lax.axis_index("s") * 2 + lax.axis_index("c")
        base = wid * CROWS
        pltpu.sync_copy(i1_hbm.at[wid], i1v)
        pltpu.sync_copy(i2_hbm.at[wid], i2v)
        pltpu.sync_copy(p1_hbm.at[pl.ds(base, CROWS)], pv1)
        pltpu.sync_copy(p2_hbm.at[pl.ds(base, CROWS)], pv2)
        ca = pltpu.async_copy(og_hbm.at[i1v], av, sa)
        cb = pltpu.async_copy(og_hbm.at[i2v], bv, sb)
        ca.wait()
        cb.wait()

        @plsc.parallel_loop(0, CROWS, 1, unroll=4)
        def body(r):
            pa = pv1[r]
            pb = pv2[r]
            for j in range(D // 16):
                av[r, pl.ds(j * 16, 16)] = (
                    av[r, pl.ds(j * 16, 16)] * pa
                    + bv[r, pl.ds(j * 16, 16)] * pb)

        pltpu.sync_copy(av, out_hbm.at[pl.ds(base, CROWS)])

    return combine


def kernel(x, Wr, W1, W2):
    b, s, d = x.shape
    x2d = x.reshape(T, D)
    pos1, pos2, p1b, p2b, te, nu = _router(x2d, Wr)
    xg = _sc_dispatch()(x2d, pos1, pos2)
    og = _ffn(te, nu, xg, W1, W2)
    out = _sc_combine()(og, pos1, pos2, p1b, p2b)
    return out.reshape(b, s, d)


# speedup vs baseline: 3.4230x; 1.0141x over previous
import functools
import math

import jax
import jax.numpy as jnp
from jax import lax
from jax.experimental import pallas as pl
from jax.experimental.pallas import tpu as pltpu
from jax.experimental.pallas import tpu_sc as plsc

T = 2048
D = 768
E = 8
K = 2
F = 2048

TILE = 128
NP = T * K + E * TILE
NT = NP // TILE

NWK = 32
CROWS = T // NWK


def _router_body(x_ref, wr_ref, pos1_ref, pos2_ref, p1_ref, p2_ref, te_ref,
                 nu_ref):
    x = x_ref[...]
    wr = wr_ref[...]
    logits = lax.dot_general(x, wr, (((1,), (1,)), ((), ())),
                             preferred_element_type=jnp.float32)
    iota = lax.broadcasted_iota(jnp.int32, (T, E), 1)
    m1 = jnp.max(logits, axis=1, keepdims=True)
    ex = jnp.exp(logits - m1)
    p = ex / jnp.sum(ex, axis=1, keepdims=True)
    a1 = jnp.min(jnp.where(logits == m1, iota, E), axis=1, keepdims=True)
    p1 = jnp.sum(jnp.where(iota == a1, p, 0.0), axis=1, keepdims=True)
    l2 = jnp.where(iota == a1, -jnp.inf, logits)
    m2 = jnp.max(l2, axis=1, keepdims=True)
    a2 = jnp.min(jnp.where(l2 == m2, iota, E), axis=1, keepdims=True)
    p2 = jnp.sum(jnp.where(iota == a2, p, 0.0), axis=1, keepdims=True)
    s = p1 + p2

    oh1 = (iota == a1).astype(jnp.float32)
    oh2 = (iota == a2).astype(jnp.float32)
    cnt1 = jnp.sum(oh1, axis=0, keepdims=True)
    cnt2 = jnp.sum(oh2, axis=0, keepdims=True)
    counts = cnt1 + cnt2
    def excl_prefix(oh):
        pre = oh
        k = 1
        while k < T:
            pre = pre + jnp.concatenate(
                [jnp.zeros((k, E), oh.dtype), pre[:T - k]], axis=0)
            k *= 2
        return pre - oh

    pre1 = excl_prefix(oh1)
    pre2 = excl_prefix(oh2)
    cap = jnp.floor((counts + (TILE - 1)) * (1.0 / TILE)) * TILE
    tri = (lax.broadcasted_iota(jnp.int32, (E, E), 0)
           <= lax.broadcasted_iota(jnp.int32, (E, E), 1)).astype(jnp.float32)
    end_pad = lax.dot_general(cap, tri, (((1,), (0,)), ((), ())),
                              preferred_element_type=jnp.float32)
    start_pad = end_pad - cap
    b1 = jnp.sum(jnp.where(iota == a1, start_pad + pre1, 0.0),
                 axis=1, keepdims=True)
    b2 = jnp.sum(jnp.where(iota == a2, start_pad + cnt1 + pre2, 0.0),
                 axis=1, keepdims=True)
    pos1_ref[...] = b1.astype(jnp.int32).reshape(NWK, CROWS)
    pos2_ref[...] = b2.astype(jnp.int32).reshape(NWK, CROWS)
    p1_ref[...] = jnp.broadcast_to(p1 / s, (T, 16))
    p2_ref[...] = jnp.broadcast_to(p2 / s, (T, 16))
    ts = lax.broadcasted_iota(jnp.int32, (NT, E), 0) * TILE
    te = jnp.sum((ts >= end_pad.astype(jnp.int32)).astype(jnp.int32),
                 axis=1, keepdims=True)
    te_ref[...] = jnp.minimum(te, E - 1)
    lane = lax.broadcasted_iota(jnp.int32, (1, E), 1)
    total = jnp.sum(jnp.where(lane == E - 1, end_pad, 0.0),
                    axis=1, keepdims=True)
    nu_ref[...] = (total * (1.0 / TILE)).astype(jnp.int32)


def _router(x2d, Wr):
    return pl.pallas_call(
        _router_body,
        out_shape=(
            jax.ShapeDtypeStruct((NWK, CROWS), jnp.int32),
            jax.ShapeDtypeStruct((NWK, CROWS), jnp.int32),
            jax.ShapeDtypeStruct((T, 16), jnp.float32),
            jax.ShapeDtypeStruct((T, 16), jnp.float32),
            jax.ShapeDtypeStruct((NT, 1), jnp.int32),
            jax.ShapeDtypeStruct((1, 1), jnp.int32),
        ),
    )(x2d, Wr)


@functools.cache
def _sc_mesh():
    return plsc.VectorSubcoreMesh(core_axis_name="c", subcore_axis_name="s")


@functools.cache
def _sc_dispatch():
    @functools.partial(
        pl.kernel,
        mesh=_sc_mesh(),
        out_type=jax.ShapeDtypeStruct((NP, D), jnp.float32),
        scratch_types=[
            pltpu.VMEM((CROWS,), jnp.int32),
            pltpu.VMEM((CROWS,), jnp.int32),
            pltpu.VMEM((CROWS, D), jnp.float32),
            pltpu.SemaphoreType.DMA,
            pltpu.SemaphoreType.DMA,
            pltpu.SemaphoreType.DMA,
        ],
    )
    def dispatch(x_hbm, pos1_hbm, pos2_hbm, xg_hbm, i1v, i2v, rows_v,
                 s0, s1, s2):
        wid = lax.axis_index("s") * 2 + lax.axis_index("c")
        base = wid * CROWS
        pltpu.sync_copy(pos1_hbm.at[wid], i1v)
        pltpu.sync_copy(pos2_hbm.at[wid], i2v)
        pltpu.async_copy(x_hbm.at[pl.ds(base, CROWS)], rows_v, s0).wait()
        c1 = pltpu.async_copy(rows_v, xg_hbm.at[i1v], s1)
        c2 = pltpu.async_copy(rows_v, xg_hbm.at[i2v], s2)
        c1.wait()
        c2.wait()

    return dispatch


def _ffn_body(te_ref, nu_ref, xg_ref, w1_ref, w2_ref, og_ref, w1b, w2b):
    i = pl.program_id(0)
    active = i < nu_ref[0, 0]
    changed = jnp.logical_and(
        active,
        jnp.logical_or(i == 0,
                       te_ref[i, 0] != te_ref[jnp.maximum(i - 1, 0), 0]))

    @pl.when(changed)
    def _():
        w1b[...] = w1_ref[0].astype(jnp.bfloat16)
        w2b[...] = w2_ref[0].astype(jnp.bfloat16)

    @pl.when(active)
    def _():
        xg = xg_ref[...].astype(jnp.bfloat16)
        h = lax.dot_general(xg, w1b[...], (((1,), (1,)), ((), ())),
                            preferred_element_type=jnp.float32)
        h = 0.5 * h * (1.0 + lax.erf(h * (1.0 / math.sqrt(2.0))))
        og_ref[...] = lax.dot_general(h.astype(jnp.bfloat16), w2b[...],
                                      (((1,), (1,)), ((), ())),
                                      preferred_element_type=jnp.float32)


def _ffn(te, nu, xg, W1, W2):
    grid_spec = pltpu.PrefetchScalarGridSpec(
        num_scalar_prefetch=2,
        grid=(NT,),
        in_specs=[
            pl.BlockSpec((TILE, D), lambda i, te, nu: (i, 0)),
            pl.BlockSpec((1, F, D), lambda i, te, nu: (te[i, 0], 0, 0)),
            pl.BlockSpec((1, D, F), lambda i, te, nu: (te[i, 0], 0, 0)),
        ],
        out_specs=pl.BlockSpec((TILE, D), lambda i, te, nu: (i, 0)),
        scratch_shapes=[
            pltpu.VMEM((F, D), jnp.bfloat16),
            pltpu.VMEM((D, F), jnp.bfloat16),
        ],
    )
    return pl.pallas_call(
        _ffn_body,
        grid_spec=grid_spec,
        out_shape=jax.ShapeDtypeStruct((NP, D), jnp.float32),
    )(te, nu, xg, W1, W2)


@functools.cache
def _sc_combine():
    @functools.partial(
        pl.kernel,
        mesh=_sc_mesh(),
        out_type=jax.ShapeDtypeStruct((T, D), jnp.float32),
        scratch_types=[
            pltpu.VMEM((CROWS,), jnp.int32),
            pltpu.VMEM((CROWS,), jnp.int32),
            pltpu.VMEM((CROWS, 16), jnp.float32),
            pltpu.VMEM((CROWS, 16), jnp.float32),
            pltpu.VMEM((CROWS, D), jnp.float32),
            pltpu.VMEM((CROWS, D), jnp.float32),
            pltpu.SemaphoreType.DMA,
            pltpu.SemaphoreType.DMA,
            pltpu.SemaphoreType.DMA,
        ],
    )
    def combine(og_hbm, i1_hbm, i2_hbm, p1_hbm, p2_hbm, out_hbm,
                i1v, i2v, pv1, pv2, av, bv, sa, sb, sc):
        wid = lax.axis_index("s") * 2 + lax.axis_index("c")
        base = wid * CROWS
        pltpu.sync_copy(i1_hbm.at[wid], i1v)
        pltpu.sync_copy(i2_hbm.at[wid], i2v)
        pltpu.sync_copy(p1_hbm.at[pl.ds(base, CROWS)], pv1)
        pltpu.sync_copy(p2_hbm.at[pl.ds(base, CROWS)], pv2)
        half = CROWS // 2
        c0 = pltpu.async_copy(og_hbm.at[i1v.at[pl.ds(0, half)]],
                              av.at[pl.ds(0, half)], sa)
        c1 = pltpu.async_copy(og_hbm.at[i2v.at[pl.ds(0, half)]],
                              bv.at[pl.ds(0, half)], sb)
        c2 = pltpu.async_copy(og_hbm.at[i1v.at[pl.ds(half, half)]],
                              av.at[pl.ds(half, half)], sa)
        c3 = pltpu.async_copy(og_hbm.at[i2v.at[pl.ds(half, half)]],
                              bv.at[pl.ds(half, half)], sb)

        def fma(lo, hi):
            @plsc.parallel_loop(lo, hi, 1, unroll=4)
            def body(r):
                pa = pv1[r]
                pb = pv2[r]
                for j in range(D // 16):
                    av[r, pl.ds(j * 16, 16)] = (
                        av[r, pl.ds(j * 16, 16)] * pa
                        + bv[r, pl.ds(j * 16, 16)] * pb)

        c0.wait()
        c1.wait()
        fma(0, half)
        p0 = pltpu.async_copy(av.at[pl.ds(0, half)],
                              out_hbm.at[pl.ds(base, half)], sc)
        c2.wait()
        c3.wait()
        fma(half, CROWS)
        pltpu.sync_copy(av.at[pl.ds(half, half)],
                        out_hbm.at[pl.ds(base + half, half)])
        p0.wait()

    return combine


def kernel(x, Wr, W1, W2):
    b, s, d = x.shape
    x2d = x.reshape(T, D)
    pos1, pos2, p1b, p2b, te, nu = _router(x2d, Wr)
    xg = _sc_dispatch()(x2d, pos1, pos2)
    og = _ffn(te, nu, xg, W1, W2)
    out = _sc_combine()(og, pos1, pos2, p1b, p2b)
    return out.reshape(b, s, d)
